# Initial kernel scaffold; baseline (speedup 1.0000x reference)
import jax
import jax.numpy as jnp
from jax.experimental import pallas as pl


def kernel(x, W):
    raise NotImplementedError("write your pallas kernel here")


# speedup vs baseline: 1.8668x; 1.8668x over previous
import functools

import jax
import jax.numpy as jnp
from jax import lax
from jax.experimental import pallas as pl
from jax.experimental.pallas import tpu as pltpu
from jax.experimental.pallas import tpu_sc as plsc

DIM = 64
B = 16384 * 50
NC = 2
NS = 16
NW = NC * NS
BPW = B // NW
CH = 128
NCHUNK = BPW // CH
NBUF = 4
NOUTER = NCHUNK // NBUF

_mesh = plsc.VectorSubcoreMesh(core_axis_name="c", subcore_axis_name="s")


@functools.partial(
    pl.kernel,
    mesh=_mesh,
    out_type=jax.ShapeDtypeStruct((B, DIM), jnp.float32),
    scratch_types=[
        pltpu.VMEM((NCHUNK, CH), jnp.int32),
        pltpu.VMEM((NBUF, CH, DIM), jnp.float32),
    ] + [pltpu.SemaphoreType.DMA] * NBUF,
    compiler_params=pltpu.CompilerParams(use_tc_tiling_on_sc=False),
)
def _embed(w_hbm, x_hbm, out_hbm, idx_v, rows_v, *sems):
    wid = lax.axis_index("s") * NC + lax.axis_index("c")
    base = wid * BPW

    pltpu.sync_copy(x_hbm.at[wid], idx_v)

    for b in range(NBUF):
        pltpu.make_async_copy(
            w_hbm.at[idx_v.at[b]], rows_v.at[b], sems[b]).start()

    def outer(g, carry):
        for b in range(NBUF):
            ch = g * NBUF + b
            pltpu.make_async_copy(
                out_hbm.at[pl.ds(0, CH)], rows_v.at[b], sems[b]).wait()
            pltpu.sync_copy(
                rows_v.at[b], out_hbm.at[pl.ds(base + ch * CH, CH)])
            pltpu.make_async_copy(
                w_hbm.at[idx_v.at[ch + NBUF]], rows_v.at[b], sems[b]).start()
        return carry

    lax.fori_loop(0, NOUTER - 1, outer, 0)

    for b in range(NBUF):
        ch = (NOUTER - 1) * NBUF + b
        pltpu.make_async_copy(
            out_hbm.at[pl.ds(0, CH)], rows_v.at[b], sems[b]).wait()
        pltpu.sync_copy(
            rows_v.at[b], out_hbm.at[pl.ds(base + ch * CH, CH)])


def kernel(x, W):
    xf = jnp.reshape(x.astype(jnp.int32), (NW, NCHUNK, CH))
    out = _embed(W, xf)
    return jnp.reshape(out, x.shape + (DIM,))
